# baseline (speedup 1.0000x reference)
import functools

import jax
import jax.numpy as jnp
import ml_dtypes as _mld
import numpy as _np
from jax import lax
from jax.experimental import pallas as pl
from jax.experimental.pallas import tpu as pltpu
from jax.experimental.pallas import tpu_sc as plsc

E = 8
DM = 1024
DF = 4096
CAP = 640
NSLOT = E * CAP
NC = 2
NS = 16
NW = NC * NS
L = 16
TSEQ = 2048
TPW = TSEQ // NW
NROW = 5248
TRASH = NSLOT
FT = 512

_TRI = _np.tril(_np.ones((TSEQ, TSEQ), _np.float32)).astype(_mld.bfloat16)


def _gate_body(x_ref, wg_ref, tri_ref, d1_ref, d2_ref, g1_ref, g2_ref,
               fill_ref, sw_ref, xmean_ref):
    xf = x_ref[...]
    T = xf.shape[0]
    logits = jnp.dot(xf, wg_ref[...], preferred_element_type=jnp.float32)
    iotaE = lax.broadcasted_iota(jnp.int32, (T, E), 1)
    l1 = jnp.max(logits, axis=-1, keepdims=True)
    e1 = jnp.min(jnp.where(logits >= l1, iotaE, E), axis=-1, keepdims=True)
    lm = jnp.where(iotaE == e1, -jnp.inf, logits)
    l2 = jnp.max(lm, axis=-1, keepdims=True)
    e2 = jnp.min(jnp.where(lm >= l2, iotaE, E), axis=-1, keepdims=True)
    g1v = 1.0 / (1.0 + jnp.exp(l2 - l1))
    g2v = 1.0 - g1v
    mask1 = (iotaE == e1).astype(jnp.float32)
    mask2 = (iotaE == e2).astype(jnp.float32)
    both = jnp.concatenate([mask1, mask2], axis=1)
    cum = jnp.dot(tri_ref[...], both.astype(jnp.bfloat16),
                  preferred_element_type=jnp.float32)
    cum1 = cum[:, :E]
    cum2 = cum[:, E:]
    n1 = cum1[T - 1:T, :]
    n2 = cum2[T - 1:T, :]
    loc1 = cum1 - 1.0
    loc2 = cum2 - 1.0 + n1
    m1k = mask1 * (loc1 < CAP).astype(jnp.float32)
    m2k = mask2 * (loc2 < CAP).astype(jnp.float32)
    pos1 = jnp.sum(loc1 * m1k, axis=-1, keepdims=True).astype(jnp.int32)
    pos2 = jnp.sum(loc2 * m2k, axis=-1, keepdims=True).astype(jnp.int32)
    kept1 = jnp.sum(m1k, axis=-1, keepdims=True)
    kept2 = jnp.sum(m2k, axis=-1, keepdims=True)
    g1 = g1v * kept1
    g2 = g2v * kept2
    d1 = e1 * CAP + pos1
    d2 = e2 * CAP + pos2
    d1_ref[...] = jnp.where(kept1 > 0.0, d1, TRASH)
    d2_ref[...] = jnp.where(kept2 > 0.0, d2, TRASH)
    g1_ref[...] = g1
    g2_ref[...] = g2
    fill_ref[...] = jnp.minimum(n1 + n2, float(CAP))
    sw_ref[...] = jnp.sum(m1k * g1 + m2k * g2, axis=0, keepdims=True)
    xmean_ref[...] = jnp.sum(xf, axis=0, keepdims=True) * (1.0 / T)


def _gate(xf, Wg, tri):
    T = xf.shape[0]
    f32, i32 = jnp.float32, jnp.int32
    outs = [((T, 1), i32), ((T, 1), i32), ((T, 1), f32), ((T, 1), f32),
            ((1, E), f32), ((1, E), f32), ((1, DM), f32)]
    return pl.pallas_call(
        _gate_body,
        out_shape=[jax.ShapeDtypeStruct(s, d) for s, d in outs],
    )(xf, Wg, tri)


def _dispatch_body(d1_hbm, d2_hbm, g1_hbm, g2_hbm, xf_hbm,
                   buf_hbm, w_hbm,
                   idx1v, idx2v, rowsv, dd1v, dd2v, gg1v, gg2v, wv,
                   sem1, sem2):
    cid = lax.axis_index("c")
    sid = lax.axis_index("s")
    wid = sid * NC + cid
    t0 = wid * TPW

    pltpu.sync_copy(xf_hbm.at[pl.ds(t0, TPW)], rowsv)
    pltpu.sync_copy(d1_hbm.at[pl.ds(t0, TPW)], idx1v)
    pltpu.sync_copy(d2_hbm.at[pl.ds(t0, TPW)], idx2v)
    cp1 = pltpu.async_copy(rowsv, buf_hbm.at[idx1v], sem1)
    cp2 = pltpu.async_copy(rowsv, buf_hbm.at[idx2v], sem2)

    @pl.when(jnp.logical_and(cid == 0, sid == 0))
    def _():
        T = dd1v.shape[0]
        pltpu.sync_copy(d1_hbm, dd1v)
        pltpu.sync_copy(d2_hbm, dd2v)
        pltpu.sync_copy(g1_hbm, gg1v)
        pltpu.sync_copy(g2_hbm, gg2v)
        zf = jnp.zeros((L,), jnp.float32)

        def zero_body(i, _):
            wv[pl.ds(i * L, L)] = zf
            return 0

        lax.fori_loop(0, NSLOT // L, zero_body, 0, unroll=4)

        def scat_body(i, _):
            sl = pl.ds(i * L, L)
            i1 = dd1v[sl]
            i2 = dd2v[sl]
            plsc.store_scatter(wv, [i1], gg1v[sl], mask=i1 < NSLOT)
            plsc.store_scatter(wv, [i2], gg2v[sl], mask=i2 < NSLOT)
            return 0

        lax.fori_loop(0, T // L, scat_body, 0, unroll=4)
        pltpu.sync_copy(wv, w_hbm)

    cp1.wait()
    cp2.wait()


def _dispatch(d1, d2, g1, g2, xf):
    T = xf.shape[0]
    mesh = plsc.VectorSubcoreMesh(core_axis_name="c", subcore_axis_name="s")
    f32, i32 = jnp.float32, jnp.int32
    kern = pl.kernel(
        _dispatch_body,
        out_type=[jax.ShapeDtypeStruct((NROW, DM), f32),
                  jax.ShapeDtypeStruct((NSLOT,), f32)],
        mesh=mesh,
        scratch_types=[
            pltpu.VMEM((TPW,), i32), pltpu.VMEM((TPW,), i32),
            pltpu.VMEM((TPW, DM), f32),
            pltpu.VMEM((T,), i32), pltpu.VMEM((T,), i32),
            pltpu.VMEM((T,), f32), pltpu.VMEM((T,), f32),
            pltpu.VMEM((NSLOT,), f32),
            pltpu.SemaphoreType.DMA, pltpu.SemaphoreType.DMA,
        ],
        compiler_params=pltpu.CompilerParams(needs_layout_passes=False),
    )
    return kern(d1, d2, g1, g2, xf)


NCH = DF // FT


def _ffn_body(buf_ref, w1_ref, b1_ref, w2d_ref, fill_ref, w2_ref, xmean_ref,
              sw_ref, b2_ref, y_ref, out_ref, xbf_ref, sent_ref):
    e = pl.program_id(0)
    c = pl.program_id(1)

    @pl.when(jnp.logical_and(e == 0, c == 0))
    def _():
        sent_ref[...] = jnp.zeros_like(sent_ref)

    rowE = lax.broadcasted_iota(jnp.int32, (1, E), 1)

    @pl.when(c == 0)
    def _():
        fe = jnp.sum(jnp.where(rowE == e, fill_ref[...], 0.0))
        rows = lax.broadcasted_iota(jnp.int32, (CAP, 1), 0)
        xbf_ref[...] = jnp.where(rows.astype(jnp.float32) < fe,
                                 buf_ref[...], 0.0).astype(jnp.bfloat16)

    @pl.when(c > 0)
    def _():
        w1c = w1_ref[0].astype(jnp.bfloat16)
        h = jnp.dot(xbf_ref[...], w1c, preferred_element_type=jnp.float32)
        rowEF = lax.broadcasted_iota(jnp.int32, (E, FT), 0)
        b1row = jnp.sum(jnp.where(rowEF == e, b1_ref[...], 0.0), axis=0,
                        keepdims=True)
        h = jnp.maximum(h + b1row, 0.0)
        rowEC = lax.broadcasted_iota(jnp.int32, (E, CAP), 0)
        we = jnp.sum(jnp.where(rowEC == e, w2d_ref[...], 0.0), axis=0,
                     keepdims=True)
        red = jnp.dot(we, h, preferred_element_type=jnp.float32)
        sent_ref[...] += jnp.dot(red, w2_ref[0],
                                 preferred_element_type=jnp.float32)

    @pl.when(jnp.logical_and(e == E - 1, c == NCH))
    def _():
        bias = jnp.dot(sw_ref[...], b2_ref[...],
                       preferred_element_type=jnp.float32)
        sent = xmean_ref[...] + (sent_ref[...] + bias) * (1.0 / TSEQ)
        mm = jnp.max(sent)
        lse = jnp.log(jnp.sum(jnp.exp(sent - mm))) + mm
        col = lax.broadcasted_iota(jnp.int32, (1, DM), 1)
        tgt = jnp.sum(jnp.where(col == y_ref[0, 0], sent, 0.0))
        out_ref[0, 0] = lse - tgt


def _ffn_loss(buf2d, W1, b1, w2d, fill, W2, xmean, sw, b2, y2):
    grid = (E, NCH + 1)
    cc = lambda c: jnp.maximum(c - 1, 0)
    return pl.pallas_call(
        _ffn_body,
        grid=grid,
        in_specs=[
            pl.BlockSpec((CAP, DM), lambda e, c: (e, 0)),
            pl.BlockSpec((1, DM, FT), lambda e, c: (e, 0, cc(c))),
            pl.BlockSpec((E, FT), lambda e, c: (0, cc(c))),
            pl.BlockSpec((E, CAP), lambda e, c: (0, 0)),
            pl.BlockSpec((1, E), lambda e, c: (0, 0)),
            pl.BlockSpec((1, FT, DM), lambda e, c: (e, cc(c), 0)),
            pl.BlockSpec((1, DM), lambda e, c: (0, 0)),
            pl.BlockSpec((1, E), lambda e, c: (0, 0)),
            pl.BlockSpec((E, DM), lambda e, c: (0, 0)),
            pl.BlockSpec(memory_space=pltpu.SMEM),
        ],
        out_specs=pl.BlockSpec(memory_space=pltpu.SMEM),
        out_shape=jax.ShapeDtypeStruct((1, 1), jnp.float32),
        scratch_shapes=[pltpu.VMEM((CAP, DM), jnp.bfloat16),
                        pltpu.VMEM((1, DM), jnp.float32)],
    )(buf2d, W1, b1, w2d, fill, W2, xmean, sw, b2, y2)


def kernel(x, y, Wg, W1, b1, W2, b2):
    B, S, _ = x.shape
    T = B * S
    xf = x.reshape(T, DM)
    tri = jnp.asarray(_TRI)
    d1, d2, g1, g2, fill, sw, xmean = _gate(xf, Wg, tri)
    buf, w = _dispatch(d1.reshape(T), d2.reshape(T),
                       g1.reshape(T), g2.reshape(T), xf)
    w2d = w.reshape(E, CAP)
    y2 = y.astype(jnp.int32).reshape(1, 1)
    loss = _ffn_loss(buf, W1, b1, w2d, fill, W2, xmean, sw, b2, y2)
    return loss.reshape(())


# speedup vs baseline: 1.1773x; 1.1773x over previous
import functools

import jax
import jax.numpy as jnp
import ml_dtypes as _mld
import numpy as _np
from jax import lax
from jax.experimental import pallas as pl
from jax.experimental.pallas import tpu as pltpu
from jax.experimental.pallas import tpu_sc as plsc

E = 8
DM = 1024
DF = 4096
CAP = 640
NSLOT = E * CAP
NC = 2
NS = 16
NW = NC * NS
L = 16
TSEQ = 2048
TPW = TSEQ // NW
NROW = 5248
TRASH = NSLOT
FT = 1024

_TRI = _np.tril(_np.ones((TSEQ, TSEQ), _np.float32)).astype(_mld.bfloat16)


def _gate_body(x_ref, wg_ref, tri_ref, d1_ref, d2_ref, g1_ref, g2_ref,
               fill_ref, sw_ref, xmean_ref, g1r_ref, g2r_ref):
    xf = x_ref[...]
    T = xf.shape[0]
    logits = jnp.dot(xf, wg_ref[...], preferred_element_type=jnp.float32)
    iotaE = lax.broadcasted_iota(jnp.int32, (T, E), 1)
    l1 = jnp.max(logits, axis=-1, keepdims=True)
    e1 = jnp.min(jnp.where(logits >= l1, iotaE, E), axis=-1, keepdims=True)
    lm = jnp.where(iotaE == e1, -jnp.inf, logits)
    l2 = jnp.max(lm, axis=-1, keepdims=True)
    e2 = jnp.min(jnp.where(lm >= l2, iotaE, E), axis=-1, keepdims=True)
    g1v = 1.0 / (1.0 + jnp.exp(l2 - l1))
    g2v = 1.0 - g1v
    mask1 = (iotaE == e1).astype(jnp.float32)
    mask2 = (iotaE == e2).astype(jnp.float32)
    both = jnp.concatenate([mask1, mask2], axis=1)
    cum = jnp.dot(tri_ref[...], both.astype(jnp.bfloat16),
                  preferred_element_type=jnp.float32)
    cum1 = cum[:, :E]
    cum2 = cum[:, E:]
    n1 = cum1[T - 1:T, :]
    n2 = cum2[T - 1:T, :]
    loc1 = cum1 - 1.0
    loc2 = cum2 - 1.0 + n1
    m1k = mask1 * (loc1 < CAP).astype(jnp.float32)
    m2k = mask2 * (loc2 < CAP).astype(jnp.float32)
    pos1 = jnp.sum(loc1 * m1k, axis=-1, keepdims=True).astype(jnp.int32)
    pos2 = jnp.sum(loc2 * m2k, axis=-1, keepdims=True).astype(jnp.int32)
    kept1 = jnp.sum(m1k, axis=-1, keepdims=True)
    kept2 = jnp.sum(m2k, axis=-1, keepdims=True)
    g1 = g1v * kept1
    g2 = g2v * kept2
    d1 = e1 * CAP + pos1
    d2 = e2 * CAP + pos2
    d1_ref[...] = jnp.where(kept1 > 0.0, d1, TRASH)
    d2_ref[...] = jnp.where(kept2 > 0.0, d2, TRASH)
    g1_ref[...] = g1
    g2_ref[...] = g2
    g1r_ref[...] = jnp.broadcast_to(g1, (T, 128))
    g2r_ref[...] = jnp.broadcast_to(g2, (T, 128))
    fill_ref[...] = jnp.minimum(n1 + n2, float(CAP))
    sw_ref[...] = jnp.sum(m1k * g1 + m2k * g2, axis=0, keepdims=True)
    xmean_ref[...] = jnp.sum(xf, axis=0, keepdims=True) * (1.0 / T)


def _gate(xf, Wg, tri):
    T = xf.shape[0]
    f32, i32 = jnp.float32, jnp.int32
    outs = [((T, 1), i32), ((T, 1), i32), ((T, 1), f32), ((T, 1), f32),
            ((1, E), f32), ((1, E), f32), ((1, DM), f32),
            ((T, 128), f32), ((T, 128), f32)]
    return pl.pallas_call(
        _gate_body,
        out_shape=[jax.ShapeDtypeStruct(s, d) for s, d in outs],
    )(xf, Wg, tri)


def _dispatch_body(d1_hbm, d2_hbm, g1r_hbm, g2r_hbm, xf_hbm,
                   buf_hbm, wbuf_hbm,
                   idx1v, idx2v, rowsv, g1v, g2v,
                   sem1, sem2, sem3, sem4):
    cid = lax.axis_index("c")
    sid = lax.axis_index("s")
    wid = sid * NC + cid
    t0 = wid * TPW

    pltpu.sync_copy(xf_hbm.at[pl.ds(t0, TPW)], rowsv)
    pltpu.sync_copy(d1_hbm.at[pl.ds(t0, TPW)], idx1v)
    pltpu.sync_copy(d2_hbm.at[pl.ds(t0, TPW)], idx2v)
    pltpu.sync_copy(g1r_hbm.at[pl.ds(t0, TPW)], g1v)
    pltpu.sync_copy(g2r_hbm.at[pl.ds(t0, TPW)], g2v)
    cp1 = pltpu.async_copy(rowsv, buf_hbm.at[idx1v], sem1)
    cp2 = pltpu.async_copy(rowsv, buf_hbm.at[idx2v], sem2)
    cp3 = pltpu.async_copy(g1v, wbuf_hbm.at[idx1v], sem3)
    cp4 = pltpu.async_copy(g2v, wbuf_hbm.at[idx2v], sem4)
    cp1.wait()
    cp2.wait()
    cp3.wait()
    cp4.wait()


def _dispatch(d1, d2, g1r, g2r, xf):
    T = xf.shape[0]
    mesh = plsc.VectorSubcoreMesh(core_axis_name="c", subcore_axis_name="s")
    f32, i32 = jnp.float32, jnp.int32
    kern = pl.kernel(
        _dispatch_body,
        out_type=[jax.ShapeDtypeStruct((NROW, DM), f32),
                  jax.ShapeDtypeStruct((NROW, 128), f32)],
        mesh=mesh,
        scratch_types=[
            pltpu.VMEM((TPW,), i32), pltpu.VMEM((TPW,), i32),
            pltpu.VMEM((TPW, DM), f32),
            pltpu.VMEM((TPW, 128), f32), pltpu.VMEM((TPW, 128), f32),
            pltpu.SemaphoreType.DMA, pltpu.SemaphoreType.DMA,
            pltpu.SemaphoreType.DMA, pltpu.SemaphoreType.DMA,
        ],
        compiler_params=pltpu.CompilerParams(needs_layout_passes=False),
    )
    return kern(d1, d2, g1r, g2r, xf)


NCH = DF // FT


def _ffn_body(buf_ref, w1_ref, b1_ref, wbuf_ref, fill_ref, w2_ref, xmean_ref,
              sw_ref, b2_ref, y_ref, out_ref, xbf_ref, wcol_ref, sent_ref):
    e = pl.program_id(0)
    c = pl.program_id(1)

    @pl.when(jnp.logical_and(e == 0, c == 0))
    def _():
        sent_ref[...] = jnp.zeros_like(sent_ref)

    rowE = lax.broadcasted_iota(jnp.int32, (1, E), 1)

    @pl.when(c == 0)
    def _():
        fe = jnp.sum(jnp.where(rowE == e, fill_ref[...], 0.0))
        rows = lax.broadcasted_iota(jnp.int32, (CAP, 1), 0)
        live = rows.astype(jnp.float32) < fe
        xbf_ref[...] = jnp.where(live, buf_ref[...], 0.0).astype(jnp.bfloat16)
        wcol_ref[...] = jnp.where(live, wbuf_ref[:, :1], 0.0)

    @pl.when(c > 0)
    def _():
        w1c = w1_ref[0].astype(jnp.bfloat16)
        h = jnp.dot(xbf_ref[...], w1c, preferred_element_type=jnp.float32)
        rowEF = lax.broadcasted_iota(jnp.int32, (E, FT), 0)
        b1row = jnp.sum(jnp.where(rowEF == e, b1_ref[...], 0.0), axis=0,
                        keepdims=True)
        h = jnp.maximum(h + b1row, 0.0)
        red = jnp.sum(h * wcol_ref[...], axis=0, keepdims=True)
        sent_ref[...] += jnp.dot(red, w2_ref[0],
                                 preferred_element_type=jnp.float32)

    @pl.when(jnp.logical_and(e == E - 1, c == NCH))
    def _():
        bias = jnp.dot(sw_ref[...], b2_ref[...],
                       preferred_element_type=jnp.float32)
        sent = xmean_ref[...] + (sent_ref[...] + bias) * (1.0 / TSEQ)
        mm = jnp.max(sent)
        lse = jnp.log(jnp.sum(jnp.exp(sent - mm))) + mm
        col = lax.broadcasted_iota(jnp.int32, (1, DM), 1)
        tgt = jnp.sum(jnp.where(col == y_ref[0, 0], sent, 0.0))
        out_ref[0, 0] = lse - tgt


def _ffn_loss(buf2d, W1, b1, wbuf, fill, W2, xmean, sw, b2, y2):
    grid = (E, NCH + 1)
    cc = lambda c: jnp.maximum(c - 1, 0)
    return pl.pallas_call(
        _ffn_body,
        grid=grid,
        in_specs=[
            pl.BlockSpec((CAP, DM), lambda e, c: (e, 0)),
            pl.BlockSpec((1, DM, FT), lambda e, c: (e, 0, cc(c))),
            pl.BlockSpec((E, FT), lambda e, c: (0, cc(c))),
            pl.BlockSpec((CAP, 128), lambda e, c: (e, 0)),
            pl.BlockSpec((1, E), lambda e, c: (0, 0)),
            pl.BlockSpec((1, FT, DM), lambda e, c: (e, cc(c), 0)),
            pl.BlockSpec((1, DM), lambda e, c: (0, 0)),
            pl.BlockSpec((1, E), lambda e, c: (0, 0)),
            pl.BlockSpec((E, DM), lambda e, c: (0, 0)),
            pl.BlockSpec(memory_space=pltpu.SMEM),
        ],
        out_specs=pl.BlockSpec(memory_space=pltpu.SMEM),
        out_shape=jax.ShapeDtypeStruct((1, 1), jnp.float32),
        scratch_shapes=[pltpu.VMEM((CAP, DM), jnp.bfloat16),
                        pltpu.VMEM((CAP, 1), jnp.float32),
                        pltpu.VMEM((1, DM), jnp.float32)],
    )(buf2d, W1, b1, wbuf, fill, W2, xmean, sw, b2, y2)


def kernel(x, y, Wg, W1, b1, W2, b2):
    B, S, _ = x.shape
    T = B * S
    xf = x.reshape(T, DM)
    tri = jnp.asarray(_TRI)
    d1, d2, g1, g2, fill, sw, xmean, g1r, g2r = _gate(xf, Wg, tri)
    buf, wbuf = _dispatch(d1.reshape(T), d2.reshape(T), g1r, g2r, xf)
    y2 = y.astype(jnp.int32).reshape(1, 1)
    loss = _ffn_loss(buf, W1, b1, wbuf, fill, W2, xmean, sw, b2, y2)
    return loss.reshape(())


# speedup vs baseline: 1.1826x; 1.0045x over previous
import functools

import jax
import jax.numpy as jnp
import ml_dtypes as _mld
import numpy as _np
from jax import lax
from jax.experimental import pallas as pl
from jax.experimental.pallas import tpu as pltpu
from jax.experimental.pallas import tpu_sc as plsc

E = 8
DM = 1024
DF = 4096
CAP = 640
NSLOT = E * CAP
NC = 2
NS = 16
NW = NC * NS
L = 16
TSEQ = 2048
TPW = TSEQ // NW
NROW = 5248
TRASH = NSLOT
FT = 2048

_TRI = _np.tril(_np.ones((TSEQ, TSEQ), _np.float32)).astype(_mld.bfloat16)


def _gate_body(x_ref, wg_ref, tri_ref, d1_ref, d2_ref, g1_ref, g2_ref,
               fill_ref, sw_ref, xmean_ref, g1r_ref, g2r_ref):
    xf = x_ref[...]
    T = xf.shape[0]
    logits = jnp.dot(xf, wg_ref[...], preferred_element_type=jnp.float32)
    iotaE = lax.broadcasted_iota(jnp.int32, (T, E), 1)
    l1 = jnp.max(logits, axis=-1, keepdims=True)
    e1 = jnp.min(jnp.where(logits >= l1, iotaE, E), axis=-1, keepdims=True)
    lm = jnp.where(iotaE == e1, -jnp.inf, logits)
    l2 = jnp.max(lm, axis=-1, keepdims=True)
    e2 = jnp.min(jnp.where(lm >= l2, iotaE, E), axis=-1, keepdims=True)
    g1v = 1.0 / (1.0 + jnp.exp(l2 - l1))
    g2v = 1.0 - g1v
    mask1 = (iotaE == e1).astype(jnp.float32)
    mask2 = (iotaE == e2).astype(jnp.float32)
    both = jnp.concatenate([mask1, mask2], axis=1)
    cum = jnp.dot(tri_ref[...], both.astype(jnp.bfloat16),
                  preferred_element_type=jnp.float32)
    cum1 = cum[:, :E]
    cum2 = cum[:, E:]
    n1 = cum1[T - 1:T, :]
    n2 = cum2[T - 1:T, :]
    loc1 = cum1 - 1.0
    loc2 = cum2 - 1.0 + n1
    m1k = mask1 * (loc1 < CAP).astype(jnp.float32)
    m2k = mask2 * (loc2 < CAP).astype(jnp.float32)
    pos1 = jnp.sum(loc1 * m1k, axis=-1, keepdims=True).astype(jnp.int32)
    pos2 = jnp.sum(loc2 * m2k, axis=-1, keepdims=True).astype(jnp.int32)
    kept1 = jnp.sum(m1k, axis=-1, keepdims=True)
    kept2 = jnp.sum(m2k, axis=-1, keepdims=True)
    g1 = g1v * kept1
    g2 = g2v * kept2
    d1 = e1 * CAP + pos1
    d2 = e2 * CAP + pos2
    d1_ref[...] = jnp.where(kept1 > 0.0, d1, TRASH)
    d2_ref[...] = jnp.where(kept2 > 0.0, d2, TRASH)
    g1_ref[...] = g1
    g2_ref[...] = g2
    g1r_ref[...] = jnp.broadcast_to(g1, (T, 128))
    g2r_ref[...] = jnp.broadcast_to(g2, (T, 128))
    fill_ref[...] = jnp.minimum(n1 + n2, float(CAP))
    sw_ref[...] = jnp.sum(m1k * g1 + m2k * g2, axis=0, keepdims=True)
    xmean_ref[...] = jnp.sum(xf, axis=0, keepdims=True) * (1.0 / T)


def _gate(xf, Wg, tri):
    T = xf.shape[0]
    f32, i32 = jnp.float32, jnp.int32
    outs = [((T, 1), i32), ((T, 1), i32), ((T, 1), f32), ((T, 1), f32),
            ((1, E), f32), ((1, E), f32), ((1, DM), f32),
            ((T, 128), f32), ((T, 128), f32)]
    return pl.pallas_call(
        _gate_body,
        out_shape=[jax.ShapeDtypeStruct(s, d) for s, d in outs],
    )(xf, Wg, tri)


def _dispatch_body(d1_hbm, d2_hbm, g1r_hbm, g2r_hbm, xf_hbm,
                   buf_hbm, wbuf_hbm,
                   idx1v, idx2v, rowsv, g1v, g2v,
                   sem1, sem2, sem3, sem4):
    cid = lax.axis_index("c")
    sid = lax.axis_index("s")
    wid = sid * NC + cid
    t0 = wid * TPW

    pltpu.sync_copy(xf_hbm.at[pl.ds(t0, TPW)], rowsv)
    pltpu.sync_copy(d1_hbm.at[pl.ds(t0, TPW)], idx1v)
    pltpu.sync_copy(d2_hbm.at[pl.ds(t0, TPW)], idx2v)
    pltpu.sync_copy(g1r_hbm.at[pl.ds(t0, TPW)], g1v)
    pltpu.sync_copy(g2r_hbm.at[pl.ds(t0, TPW)], g2v)
    cp1 = pltpu.async_copy(rowsv, buf_hbm.at[idx1v], sem1)
    cp2 = pltpu.async_copy(rowsv, buf_hbm.at[idx2v], sem2)
    cp3 = pltpu.async_copy(g1v, wbuf_hbm.at[idx1v], sem3)
    cp4 = pltpu.async_copy(g2v, wbuf_hbm.at[idx2v], sem4)
    cp1.wait()
    cp2.wait()
    cp3.wait()
    cp4.wait()


def _dispatch(d1, d2, g1r, g2r, xf):
    T = xf.shape[0]
    mesh = plsc.VectorSubcoreMesh(core_axis_name="c", subcore_axis_name="s")
    f32, i32 = jnp.float32, jnp.int32
    kern = pl.kernel(
        _dispatch_body,
        out_type=[jax.ShapeDtypeStruct((NROW, DM), f32),
                  jax.ShapeDtypeStruct((NROW, 128), f32)],
        mesh=mesh,
        scratch_types=[
            pltpu.VMEM((TPW,), i32), pltpu.VMEM((TPW,), i32),
            pltpu.VMEM((TPW, DM), f32),
            pltpu.VMEM((TPW, 128), f32), pltpu.VMEM((TPW, 128), f32),
            pltpu.SemaphoreType.DMA, pltpu.SemaphoreType.DMA,
            pltpu.SemaphoreType.DMA, pltpu.SemaphoreType.DMA,
        ],
        compiler_params=pltpu.CompilerParams(needs_layout_passes=False),
    )
    return kern(d1, d2, g1r, g2r, xf)


NCH = DF // FT


def _ffn_body(buf_ref, w1_ref, b1_ref, wbuf_ref, fill_ref, w2_ref, xmean_ref,
              sw_ref, b2_ref, y_ref, out_ref, xbf_ref, wcol_ref, sent_ref):
    e = pl.program_id(0)
    c = pl.program_id(1)

    @pl.when(jnp.logical_and(e == 0, c == 0))
    def _():
        sent_ref[...] = jnp.zeros_like(sent_ref)

    rowE = lax.broadcasted_iota(jnp.int32, (1, E), 1)

    @pl.when(c == 0)
    def _():
        fe = jnp.sum(jnp.where(rowE == e, fill_ref[...], 0.0))
        rows = lax.broadcasted_iota(jnp.int32, (CAP, 1), 0)
        live = rows.astype(jnp.float32) < fe
        xbf_ref[...] = jnp.where(live, buf_ref[...], 0.0).astype(jnp.bfloat16)
        wcol_ref[...] = jnp.where(live, wbuf_ref[:, :1], 0.0)

    @pl.when(c > 0)
    def _():
        w1c = w1_ref[0].astype(jnp.bfloat16)
        h = jnp.dot(xbf_ref[...], w1c, preferred_element_type=jnp.float32)
        rowEF = lax.broadcasted_iota(jnp.int32, (E, FT), 0)
        b1row = jnp.sum(jnp.where(rowEF == e, b1_ref[...], 0.0), axis=0,
                        keepdims=True)
        h = jnp.maximum(h + b1row, 0.0)
        red = jnp.sum(h * wcol_ref[...], axis=0, keepdims=True)
        sent_ref[...] += jnp.dot(red, w2_ref[0],
                                 preferred_element_type=jnp.float32)

    @pl.when(jnp.logical_and(e == E - 1, c == NCH))
    def _():
        bias = jnp.dot(sw_ref[...], b2_ref[...],
                       preferred_element_type=jnp.float32)
        sent = xmean_ref[...] + (sent_ref[...] + bias) * (1.0 / TSEQ)
        mm = jnp.max(sent)
        lse = jnp.log(jnp.sum(jnp.exp(sent - mm))) + mm
        col = lax.broadcasted_iota(jnp.int32, (1, DM), 1)
        tgt = jnp.sum(jnp.where(col == y_ref[0, 0], sent, 0.0))
        out_ref[0, 0] = lse - tgt


def _ffn_loss(buf2d, W1, b1, wbuf, fill, W2, xmean, sw, b2, y2):
    grid = (E, NCH + 1)
    cc = lambda c: jnp.maximum(c - 1, 0)
    return pl.pallas_call(
        _ffn_body,
        grid=grid,
        in_specs=[
            pl.BlockSpec((CAP, DM), lambda e, c: (e, 0)),
            pl.BlockSpec((1, DM, FT), lambda e, c: (e, 0, cc(c))),
            pl.BlockSpec((E, FT), lambda e, c: (0, cc(c))),
            pl.BlockSpec((CAP, 128), lambda e, c: (e, 0)),
            pl.BlockSpec((1, E), lambda e, c: (0, 0)),
            pl.BlockSpec((1, FT, DM), lambda e, c: (e, cc(c), 0)),
            pl.BlockSpec((1, DM), lambda e, c: (0, 0)),
            pl.BlockSpec((1, E), lambda e, c: (0, 0)),
            pl.BlockSpec((E, DM), lambda e, c: (0, 0)),
            pl.BlockSpec(memory_space=pltpu.SMEM),
        ],
        out_specs=pl.BlockSpec(memory_space=pltpu.SMEM),
        out_shape=jax.ShapeDtypeStruct((1, 1), jnp.float32),
        compiler_params=pltpu.CompilerParams(
            vmem_limit_bytes=120 * 1024 * 1024),
        scratch_shapes=[pltpu.VMEM((CAP, DM), jnp.bfloat16),
                        pltpu.VMEM((CAP, 1), jnp.float32),
                        pltpu.VMEM((1, DM), jnp.float32)],
    )(buf2d, W1, b1, wbuf, fill, W2, xmean, sw, b2, y2)


def kernel(x, y, Wg, W1, b1, W2, b2):
    B, S, _ = x.shape
    T = B * S
    xf = x.reshape(T, DM)
    tri = jnp.asarray(_TRI)
    d1, d2, g1, g2, fill, sw, xmean, g1r, g2r = _gate(xf, Wg, tri)
    buf, wbuf = _dispatch(d1.reshape(T), d2.reshape(T), g1r, g2r, xf)
    y2 = y.astype(jnp.int32).reshape(1, 1)
    loss = _ffn_loss(buf, W1, b1, wbuf, fill, W2, xmean, sw, b2, y2)
    return loss.reshape(())


# speedup vs baseline: 1.1828x; 1.0002x over previous
import jax
import jax.numpy as jnp
import ml_dtypes as _mld
import numpy as _np
from jax import lax
from jax.experimental import pallas as pl
from jax.experimental.pallas import tpu as pltpu
from jax.experimental.pallas import tpu_sc as plsc

E = 8
DM = 1024
DF = 4096
CAP = 640
NSLOT = E * CAP
NC = 2
NS = 16
NW = NC * NS
L = 16
TSEQ = 2048
TPW = TSEQ // NW
NROW = 5248
TRASH = NSLOT
FT = 2048

_TRI = _np.tril(_np.ones((TSEQ, TSEQ), _np.float32)).astype(_mld.bfloat16)


def _gate_body(x_ref, wg_ref, tri_ref, d1_ref, d2_ref, g1_ref, g2_ref,
               fill_ref, sw_ref, xmean_ref, g1r_ref, g2r_ref):
    xf = x_ref[...]
    T = xf.shape[0]
    logits = jnp.dot(xf, wg_ref[...], preferred_element_type=jnp.float32)
    iotaE = lax.broadcasted_iota(jnp.int32, (T, E), 1)
    l1 = jnp.max(logits, axis=-1, keepdims=True)
    e1 = jnp.min(jnp.where(logits >= l1, iotaE, E), axis=-1, keepdims=True)
    lm = jnp.where(iotaE == e1, -jnp.inf, logits)
    l2 = jnp.max(lm, axis=-1, keepdims=True)
    e2 = jnp.min(jnp.where(lm >= l2, iotaE, E), axis=-1, keepdims=True)
    g1v = 1.0 / (1.0 + jnp.exp(l2 - l1))
    g2v = 1.0 - g1v
    mask1 = (iotaE == e1).astype(jnp.float32)
    mask2 = (iotaE == e2).astype(jnp.float32)
    both = jnp.concatenate([mask1, mask2], axis=1)
    cum = jnp.dot(tri_ref[...], both.astype(jnp.bfloat16),
                  preferred_element_type=jnp.float32)
    cum1 = cum[:, :E]
    cum2 = cum[:, E:]
    n1 = cum1[T - 1:T, :]
    n2 = cum2[T - 1:T, :]
    loc1 = cum1 - 1.0
    loc2 = cum2 - 1.0 + n1
    m1k = mask1 * (loc1 < CAP).astype(jnp.float32)
    m2k = mask2 * (loc2 < CAP).astype(jnp.float32)
    pos1 = jnp.sum(loc1 * m1k, axis=-1, keepdims=True).astype(jnp.int32)
    pos2 = jnp.sum(loc2 * m2k, axis=-1, keepdims=True).astype(jnp.int32)
    kept1 = jnp.sum(m1k, axis=-1, keepdims=True)
    kept2 = jnp.sum(m2k, axis=-1, keepdims=True)
    g1 = g1v * kept1
    g2 = g2v * kept2
    d1 = e1 * CAP + pos1
    d2 = e2 * CAP + pos2
    d1_ref[...] = jnp.where(kept1 > 0.0, d1, TRASH)
    d2_ref[...] = jnp.where(kept2 > 0.0, d2, TRASH)
    g1_ref[...] = g1
    g2_ref[...] = g2
    g1r_ref[...] = jnp.broadcast_to(g1, (T, 128))
    g2r_ref[...] = jnp.broadcast_to(g2, (T, 128))
    fill_ref[...] = jnp.minimum(n1 + n2, float(CAP))
    sw_ref[...] = jnp.sum(m1k * g1 + m2k * g2, axis=0, keepdims=True)
    xmean_ref[...] = jnp.sum(xf, axis=0, keepdims=True) * (1.0 / T)


def _gate(xf, Wg, tri):
    T = xf.shape[0]
    f32, i32 = jnp.float32, jnp.int32
    outs = [((T, 1), i32), ((T, 1), i32), ((T, 1), f32), ((T, 1), f32),
            ((1, E), f32), ((1, E), f32), ((1, DM), f32),
            ((T, 128), f32), ((T, 128), f32)]
    return pl.pallas_call(
        _gate_body,
        out_shape=[jax.ShapeDtypeStruct(s, d) for s, d in outs],
    )(xf, Wg, tri)


def _dispatch_body(d1_hbm, d2_hbm, g1r_hbm, g2r_hbm, xf_hbm,
                   buf_hbm, wbuf_hbm,
                   idx1v, idx2v, rowsv, g1v, g2v,
                   sem1, sem2, sem3, sem4):
    cid = lax.axis_index("c")
    sid = lax.axis_index("s")
    wid = sid * NC + cid
    t0 = wid * TPW

    pltpu.sync_copy(xf_hbm.at[pl.ds(t0, TPW)], rowsv)
    pltpu.sync_copy(d1_hbm.at[pl.ds(t0, TPW)], idx1v)
    pltpu.sync_copy(d2_hbm.at[pl.ds(t0, TPW)], idx2v)
    pltpu.sync_copy(g1r_hbm.at[pl.ds(t0, TPW)], g1v)
    pltpu.sync_copy(g2r_hbm.at[pl.ds(t0, TPW)], g2v)
    cp1 = pltpu.async_copy(rowsv, buf_hbm.at[idx1v], sem1)
    cp2 = pltpu.async_copy(rowsv, buf_hbm.at[idx2v], sem2)
    cp3 = pltpu.async_copy(g1v, wbuf_hbm.at[idx1v], sem3)
    cp4 = pltpu.async_copy(g2v, wbuf_hbm.at[idx2v], sem4)
    cp1.wait()
    cp2.wait()
    cp3.wait()
    cp4.wait()


def _dispatch(d1, d2, g1r, g2r, xf):
    T = xf.shape[0]
    mesh = plsc.VectorSubcoreMesh(core_axis_name="c", subcore_axis_name="s")
    f32, i32 = jnp.float32, jnp.int32
    kern = pl.kernel(
        _dispatch_body,
        out_type=[jax.ShapeDtypeStruct((NROW, DM), f32),
                  jax.ShapeDtypeStruct((NROW, 128), f32)],
        mesh=mesh,
        scratch_types=[
            pltpu.VMEM((TPW,), i32), pltpu.VMEM((TPW,), i32),
            pltpu.VMEM((TPW, DM), f32),
            pltpu.VMEM((TPW, 128), f32), pltpu.VMEM((TPW, 128), f32),
            pltpu.SemaphoreType.DMA, pltpu.SemaphoreType.DMA,
            pltpu.SemaphoreType.DMA, pltpu.SemaphoreType.DMA,
        ],
        compiler_params=pltpu.CompilerParams(needs_layout_passes=False),
    )
    return kern(d1, d2, g1r, g2r, xf)


NCH = DF // FT


def _ffn_body(buf_ref, w1_ref, b1_ref, wbuf_ref, fill_ref, w2_ref, xmean_ref,
              sw_ref, b2_ref, y_ref, out_ref, xbf_ref, wcol_ref, sent_ref):
    e = pl.program_id(0)
    c = pl.program_id(1)

    @pl.when(jnp.logical_and(e == 0, c == 0))
    def _():
        sent_ref[...] = jnp.zeros_like(sent_ref)

    rowE = lax.broadcasted_iota(jnp.int32, (1, E), 1)

    @pl.when(c == 0)
    def _():
        fe = jnp.sum(jnp.where(rowE == e, fill_ref[...], 0.0))
        rows = lax.broadcasted_iota(jnp.int32, (CAP, 1), 0)
        live = rows.astype(jnp.float32) < fe
        xbf_ref[...] = jnp.where(live, buf_ref[...], 0.0).astype(jnp.bfloat16)
        wcol_ref[...] = jnp.where(live, wbuf_ref[:, :1], 0.0)

    @pl.when(c > 0)
    def _():
        w1c = w1_ref[0].astype(jnp.bfloat16)
        h = jnp.dot(xbf_ref[...], w1c, preferred_element_type=jnp.float32)
        rowEF = lax.broadcasted_iota(jnp.int32, (E, FT), 0)
        b1row = jnp.sum(jnp.where(rowEF == e, b1_ref[...], 0.0), axis=0,
                        keepdims=True)
        h = jnp.maximum(h + b1row, 0.0)
        red = jnp.sum(h * wcol_ref[...], axis=0, keepdims=True)
        sent_ref[...] += jnp.dot(red, w2_ref[0],
                                 preferred_element_type=jnp.float32)

    @pl.when(jnp.logical_and(e == E - 1, c == NCH))
    def _():
        bias = jnp.dot(sw_ref[...], b2_ref[...],
                       preferred_element_type=jnp.float32)
        sent = xmean_ref[...] + (sent_ref[...] + bias) * (1.0 / TSEQ)
        mm = jnp.max(sent)
        lse = jnp.log(jnp.sum(jnp.exp(sent - mm))) + mm
        col = lax.broadcasted_iota(jnp.int32, (1, DM), 1)
        tgt = jnp.sum(jnp.where(col == y_ref[0, 0], sent, 0.0))
        out_ref[0, 0] = lse - tgt


def _ffn_loss(buf2d, W1, b1, wbuf, fill, W2, xmean, sw, b2, y2):
    grid = (E, NCH + 1)
    cc = lambda c: jnp.maximum(c - 1, 0)
    return pl.pallas_call(
        _ffn_body,
        grid=grid,
        in_specs=[
            pl.BlockSpec((CAP, DM), lambda e, c: (e, 0)),
            pl.BlockSpec((1, DM, FT), lambda e, c: (e, 0, cc(c))),
            pl.BlockSpec((E, FT), lambda e, c: (0, cc(c))),
            pl.BlockSpec((CAP, 128), lambda e, c: (e, 0)),
            pl.BlockSpec((1, E), lambda e, c: (0, 0)),
            pl.BlockSpec((1, FT, DM), lambda e, c: (e, cc(c), 0)),
            pl.BlockSpec((1, DM), lambda e, c: (0, 0)),
            pl.BlockSpec((1, E), lambda e, c: (0, 0)),
            pl.BlockSpec((E, DM), lambda e, c: (0, 0)),
            pl.BlockSpec(memory_space=pltpu.SMEM),
        ],
        out_specs=pl.BlockSpec(memory_space=pltpu.SMEM),
        out_shape=jax.ShapeDtypeStruct((1, 1), jnp.float32),
        compiler_params=pltpu.CompilerParams(
            vmem_limit_bytes=120 * 1024 * 1024),
        scratch_shapes=[pltpu.VMEM((CAP, DM), jnp.bfloat16),
                        pltpu.VMEM((CAP, 1), jnp.float32),
                        pltpu.VMEM((1, DM), jnp.float32)],
    )(buf2d, W1, b1, wbuf, fill, W2, xmean, sw, b2, y2)


def kernel(x, y, Wg, W1, b1, W2, b2):
    B, S, _ = x.shape
    T = B * S
    xf = x.reshape(T, DM)
    tri = jnp.asarray(_TRI)
    d1, d2, g1, g2, fill, sw, xmean, g1r, g2r = _gate(xf, Wg, tri)
    buf, wbuf = _dispatch(d1.reshape(T), d2.reshape(T), g1r, g2r, xf)
    y2 = y.astype(jnp.int32).reshape(1, 1)
    loss = _ffn_loss(buf, W1, b1, wbuf, fill, W2, xmean, sw, b2, y2)
    return loss.reshape(())
